# baseline (speedup 1.0000x reference)
import functools

import jax
import jax.numpy as jnp
import numpy as np
from jax import lax
from jax.experimental import pallas as pl
from jax.experimental.pallas import tpu as pltpu
from jax.experimental.pallas import tpu_sc as plsc

N = 10000
E = 320000
HID = 128
FIL = 128
NG = 50

NC = 2
NS = 16
NW = NC * NS
EH = E // 2
EPW = EH // NW
C = 40
NCH = EPW // C
NSLAB = 5
CPS = NCH // NSLAB
PAIRS = CPS // 2
RB = 40
NRB = N // RB
MAXT = (NRB + NS - 1) // NS

def _mlp2_body(pack_out, x_ref, w1_ref, b1_ref, w2_ref, b2_ref, o_ref):
    t = jnp.dot(x_ref[...], w1_ref[...], preferred_element_type=jnp.float32)
    t = t + b1_ref[...]
    t = t * jax.lax.logistic(t)
    t = t.astype(w2_ref.dtype)
    o = jnp.dot(t, w2_ref[...], preferred_element_type=jnp.float32) + b2_ref[...]
    if not pack_out:
        o_ref[...] = o
        return
    f2 = o.shape[-1]
    a = jax.lax.bitcast_convert_type(o[:, : f2 // 2].astype(jnp.bfloat16), jnp.uint16)
    b = jax.lax.bitcast_convert_type(o[:, f2 // 2 :].astype(jnp.bfloat16), jnp.uint16)
    word = a.astype(jnp.uint32) | (b.astype(jnp.uint32) << 16)
    o_ref[...] = jax.lax.bitcast_convert_type(word, jnp.float32)


def _mlp2(x, w1, b1, w2, b2, bm, pack_out):
    m, k = x.shape
    f1 = w1.shape[1]
    f2 = w2.shape[1]
    fo = f2 // 2 if pack_out else f2
    return pl.pallas_call(
        functools.partial(_mlp2_body, pack_out),
        grid=(m // bm,),
        in_specs=[
            pl.BlockSpec((bm, k), lambda i: (i, 0)),
            pl.BlockSpec((k, f1), lambda i: (0, 0)),
            pl.BlockSpec((1, f1), lambda i: (0, 0)),
            pl.BlockSpec((f1, f2), lambda i: (0, 0)),
            pl.BlockSpec((1, f2), lambda i: (0, 0)),
        ],
        out_specs=pl.BlockSpec((bm, fo), lambda i: (i, 0)),
        out_shape=jax.ShapeDtypeStruct((m, fo), jnp.float32),
    )(x, w1, b1, w2, b2)


def _out_mlp_body(h_ref, agg_ref, aggb_ref, w1_ref, b1_ref, w2_ref, b2_ref, o_ref):
    agg = (agg_ref[0] + agg_ref[1]) + (aggb_ref[0] + aggb_ref[1])
    t = jnp.dot(agg, w1_ref[...], preferred_element_type=jnp.float32) + b1_ref[...]
    t = t * jax.lax.logistic(t)
    o_ref[...] = h_ref[...] + jnp.dot(t, w2_ref[...], preferred_element_type=jnp.float32) + b2_ref[...]


def _out_mlp(h, agg2, agg2b, w1, b1, w2, b2, bm):
    m = h.shape[0]
    return pl.pallas_call(
        _out_mlp_body,
        grid=(m // bm,),
        in_specs=[
            pl.BlockSpec((bm, HID), lambda i: (i, 0)),
            pl.BlockSpec((NC, bm, FIL), lambda i: (0, i, 0)),
            pl.BlockSpec((NC, bm, FIL), lambda i: (0, i, 0)),
            pl.BlockSpec((FIL, HID), lambda i: (0, 0)),
            pl.BlockSpec((1, HID), lambda i: (0, 0)),
            pl.BlockSpec((HID, HID), lambda i: (0, 0)),
            pl.BlockSpec((1, HID), lambda i: (0, 0)),
        ],
        out_specs=pl.BlockSpec((bm, HID), lambda i: (i, 0)),
        out_shape=jax.ShapeDtypeStruct((m, HID), jnp.float32),
    )(h, agg2, agg2b, w1, b1, w2, b2)


@functools.partial(
    pl.kernel,
    mesh=plsc.VectorSubcoreMesh(core_axis_name="c", subcore_axis_name="s"),
    compiler_params=pltpu.CompilerParams(needs_layout_passes=False),
    out_type=jax.ShapeDtypeStruct((NC, N, FIL), jnp.float32),
    scratch_types=[
        pltpu.VMEM((CPS, C), jnp.int32),
        pltpu.VMEM((CPS, C), jnp.int32),
        pltpu.VMEM((C, FIL), jnp.float32),
        pltpu.VMEM((C, FIL), jnp.float32),
        pltpu.VMEM((C, FIL // 2), jnp.float32),
        pltpu.VMEM((C, FIL // 2), jnp.float32),
        pltpu.VMEM((C, FIL), jnp.float32),
        pltpu.VMEM((C, FIL), jnp.float32),
        pltpu.VMEM_SHARED((N, FIL), jnp.float32),
        pltpu.SemaphoreType.DMA,
        pltpu.SemaphoreType.DMA,
        pltpu.SemaphoreType.DMA,
        pltpu.SemaphoreType.DMA,
        pltpu.SemaphoreType.DMA,
        pltpu.SemaphoreType.DMA,
    ],
)
def _sc_agg(x_hbm, w_hbm, src_hbm, dst_hbm, out_hbm,
            src_v, dst_v, gx0, gx1, gw0, gw1, mb0, mb1, agg_sh,
            sem_g0, sem_g1, sem_w0, sem_w1, sem_s0, sem_s1):
    c = lax.axis_index("c")
    s = lax.axis_index("s")
    wid = s * NC + c

    def _wslice(sl, j):
        return w_hbm.at[pl.ds(((wid * NSLAB + sl) * CPS + j) * C, C)]

    def _mul(gx, gw, mb):
        half = FIL // 2
        def _row(i, carry):
            for g in range(half // 16):
                wv = plsc.bitcast(gw[i, pl.ds(16 * g, 16)], jnp.bfloat16)
                wa, wb = plsc.unpack(wv, format=plsc.PackFormat.INTERLEAVED)
                mb[i, pl.ds(16 * g, 16)] = wa * gx[i, pl.ds(16 * g, 16)]
                mb[i, pl.ds(half + 16 * g, 16)] = wb * gx[i, pl.ds(half + 16 * g, 16)]
            return carry

        lax.fori_loop(0, C, _row, 0)

    def _zero_buf(i, carry):
        for cc in range(FIL // 16):
            mb0[i, pl.ds(cc * 16, 16)] = jnp.zeros((16,), jnp.float32)
        return carry

    lax.fori_loop(0, RB, _zero_buf, 0)

    def _zero_stripe(t, carry):
        idx = s + t * NS

        @pl.when(idx < NRB)
        def _():
            pltpu.sync_copy(mb0, agg_sh.at[pl.ds(idx * RB, RB)])

        return carry

    lax.fori_loop(0, MAXT, _zero_stripe, 0)
    plsc.subcore_barrier()

    def _slab(sl, carry):
        pltpu.sync_copy(src_hbm.at[wid, sl], src_v)
        pltpu.sync_copy(dst_hbm.at[wid, sl], dst_v)
        pltpu.async_copy(x_hbm.at[src_v.at[0]], gx0, sem_g0)
        pltpu.async_copy(_wslice(sl, 0), gw0, sem_w0)

        def _pair(p, carry1):
            j0 = 2 * p
            j1 = 2 * p + 1

            @pl.when(p > 0)
            def _():
                pltpu.make_async_copy(mb1, agg_sh.at[dst_v.at[0]], sem_s1).wait()

            pltpu.async_copy(x_hbm.at[src_v.at[j1]], gx1, sem_g1)
            pltpu.async_copy(_wslice(sl, j1), gw1, sem_w1)

            pltpu.make_async_copy(x_hbm.at[src_v.at[j0]], gx0, sem_g0).wait()
            pltpu.make_async_copy(_wslice(sl, j0), gw0, sem_w0).wait()
            _mul(gx0, gw0, mb0)
            pltpu.async_copy(mb0, agg_sh.at[dst_v.at[j0]], sem_s0, add=True)

            pltpu.make_async_copy(x_hbm.at[src_v.at[j1]], gx1, sem_g1).wait()
            pltpu.make_async_copy(_wslice(sl, j1), gw1, sem_w1).wait()
            _mul(gx1, gw1, mb1)
            pltpu.async_copy(mb1, agg_sh.at[dst_v.at[j1]], sem_s1, add=True)

            @pl.when(p + 1 < PAIRS)
            def _():
                pltpu.make_async_copy(mb0, agg_sh.at[dst_v.at[0]], sem_s0).wait()
                pltpu.async_copy(x_hbm.at[src_v.at[j0 + 2]], gx0, sem_g0)
                pltpu.async_copy(_wslice(sl, j0 + 2), gw0, sem_w0)

            return carry1

        lax.fori_loop(0, PAIRS, _pair, 0)
        if CPS % 2:
            jt = CPS - 1
            pltpu.make_async_copy(mb0, agg_sh.at[dst_v.at[0]], sem_s0).wait()
            pltpu.async_copy(x_hbm.at[src_v.at[jt]], gx0, sem_g0)
            pltpu.async_copy(_wslice(sl, jt), gw0, sem_w0)
            pltpu.make_async_copy(x_hbm.at[src_v.at[jt]], gx0, sem_g0).wait()
            pltpu.make_async_copy(_wslice(sl, jt), gw0, sem_w0).wait()
            _mul(gx0, gw0, mb0)
            pltpu.async_copy(mb0, agg_sh.at[dst_v.at[jt]], sem_s0, add=True)
        pltpu.make_async_copy(mb0, agg_sh.at[dst_v.at[0]], sem_s0).wait()
        pltpu.make_async_copy(mb1, agg_sh.at[dst_v.at[0]], sem_s1).wait()
        return carry

    lax.fori_loop(0, NSLAB, _slab, 0)
    plsc.subcore_barrier()

    def _writeback(t, carry):
        idx = s + t * NS

        @pl.when(idx < NRB)
        def _():
            row = idx * RB
            pltpu.sync_copy(agg_sh.at[pl.ds(row, RB)], mb0)
            pltpu.sync_copy(mb0, out_hbm.at[c, pl.ds(row, RB)])

        return carry

    lax.fori_loop(0, MAXT, _writeback, 0)


def kernel(h, edge_index, dist_feat, fw1, fb1, fw2, fb2, aw1, ab1, aw2, ab2, ow1, ob1, ow2, ob2):
    src = edge_index[0].astype(jnp.int32)
    dst = edge_index[1].astype(jnp.int32)
    src_a = src[:EH].reshape(NW, NSLAB, CPS, C)
    dst_a = dst[:EH].reshape(NW, NSLAB, CPS, C)
    src_b = src[EH:].reshape(NW, NSLAB, CPS, C)
    dst_b = dst[EH:].reshape(NW, NSLAB, CPS, C)

    bf = jnp.bfloat16
    fb1r = fb1.reshape(1, -1)
    fb2r = fb2.reshape(1, -1)
    x = _mlp2(h.astype(bf), aw1.astype(bf), ab1.reshape(1, -1),
              aw2.astype(bf), ab2.reshape(1, -1), bm=2000, pack_out=False)
    fw1b = fw1.astype(bf)
    fw2b = fw2.astype(bf)
    w_a = _mlp2(dist_feat[:EH].astype(bf), fw1b, fb1r, fw2b, fb2r,
                bm=5000, pack_out=True)
    agg2a = _sc_agg(x, w_a, src_a, dst_a)
    w_b = _mlp2(dist_feat[EH:].astype(bf), fw1b, fb1r, fw2b, fb2r,
                bm=5000, pack_out=True)
    agg2b = _sc_agg(x, w_b, src_b, dst_b)

    return _out_mlp(h, agg2a, agg2b, ow1, ob1.reshape(1, -1), ow2, ob2.reshape(1, -1), bm=1000)


# speedup vs baseline: 1.0889x; 1.0889x over previous
import functools

import jax
import jax.numpy as jnp
from jax import lax
from jax.experimental import pallas as pl
from jax.experimental.pallas import tpu as pltpu
from jax.experimental.pallas import tpu_sc as plsc

N = 10000
E = 320000
HID = 128
FIL = 128
NG = 50

NC = 2
NS = 16
NW = NC * NS
EPW = E // NW
C = 80
NCH = EPW // C
NSLAB = 5
CPS = NCH // NSLAB
PAIRS = CPS // 2
RB = 80
NRB = N // RB
MAXT = (NRB + NS - 1) // NS


def _mlp2_body(pack_out, x_ref, w1_ref, b1_ref, w2_ref, b2_ref, o_ref):
    t = jnp.dot(x_ref[...], w1_ref[...], preferred_element_type=jnp.float32)
    t = t + b1_ref[...]
    t = t * jax.lax.logistic(t)
    t = t.astype(w2_ref.dtype)
    o = jnp.dot(t, w2_ref[...], preferred_element_type=jnp.float32) + b2_ref[...]
    if not pack_out:
        o_ref[...] = o
        return
    f2 = o.shape[-1]
    a = jax.lax.bitcast_convert_type(o[:, : f2 // 2].astype(jnp.bfloat16), jnp.uint16)
    b = jax.lax.bitcast_convert_type(o[:, f2 // 2 :].astype(jnp.bfloat16), jnp.uint16)
    word = a.astype(jnp.uint32) | (b.astype(jnp.uint32) << 16)
    o_ref[...] = jax.lax.bitcast_convert_type(word, jnp.float32)


def _mlp2(x, w1, b1, w2, b2, bm, pack_out):
    m, k = x.shape
    f1 = w1.shape[1]
    f2 = w2.shape[1]
    fo = f2 // 2 if pack_out else f2
    return pl.pallas_call(
        functools.partial(_mlp2_body, pack_out),
        grid=(m // bm,),
        in_specs=[
            pl.BlockSpec((bm, k), lambda i: (i, 0)),
            pl.BlockSpec((k, f1), lambda i: (0, 0)),
            pl.BlockSpec((1, f1), lambda i: (0, 0)),
            pl.BlockSpec((f1, f2), lambda i: (0, 0)),
            pl.BlockSpec((1, f2), lambda i: (0, 0)),
        ],
        out_specs=pl.BlockSpec((bm, fo), lambda i: (i, 0)),
        out_shape=jax.ShapeDtypeStruct((m, fo), jnp.float32),
    )(x, w1, b1, w2, b2)


def _out_mlp_body(h_ref, agg_ref, w1_ref, b1_ref, w2_ref, b2_ref, o_ref):
    agg = agg_ref[0] + agg_ref[1]
    t = jnp.dot(agg, w1_ref[...], preferred_element_type=jnp.float32) + b1_ref[...]
    t = t * jax.lax.logistic(t)
    o_ref[...] = h_ref[...] + jnp.dot(t, w2_ref[...], preferred_element_type=jnp.float32) + b2_ref[...]


def _out_mlp(h, agg2, w1, b1, w2, b2, bm):
    m = h.shape[0]
    return pl.pallas_call(
        _out_mlp_body,
        grid=(m // bm,),
        in_specs=[
            pl.BlockSpec((bm, HID), lambda i: (i, 0)),
            pl.BlockSpec((NC, bm, FIL), lambda i: (0, i, 0)),
            pl.BlockSpec((FIL, HID), lambda i: (0, 0)),
            pl.BlockSpec((1, HID), lambda i: (0, 0)),
            pl.BlockSpec((HID, HID), lambda i: (0, 0)),
            pl.BlockSpec((1, HID), lambda i: (0, 0)),
        ],
        out_specs=pl.BlockSpec((bm, HID), lambda i: (i, 0)),
        out_shape=jax.ShapeDtypeStruct((m, HID), jnp.float32),
    )(h, agg2, w1, b1, w2, b2)


@functools.partial(
    pl.kernel,
    mesh=plsc.VectorSubcoreMesh(core_axis_name="c", subcore_axis_name="s"),
    compiler_params=pltpu.CompilerParams(needs_layout_passes=False),
    out_type=jax.ShapeDtypeStruct((NC, N, FIL), jnp.float32),
    scratch_types=[
        pltpu.VMEM((CPS, C), jnp.int32),
        pltpu.VMEM((CPS, C), jnp.int32),
        pltpu.VMEM((C, FIL), jnp.float32),
        pltpu.VMEM((C, FIL), jnp.float32),
        pltpu.VMEM((C, FIL // 2), jnp.float32),
        pltpu.VMEM((C, FIL // 2), jnp.float32),
        pltpu.VMEM_SHARED((N, FIL), jnp.float32),
        pltpu.SemaphoreType.DMA,
        pltpu.SemaphoreType.DMA,
        pltpu.SemaphoreType.DMA,
        pltpu.SemaphoreType.DMA,
        pltpu.SemaphoreType.DMA,
        pltpu.SemaphoreType.DMA,
    ],
)
def _sc_agg(x_hbm, w_hbm, src_hbm, dst_hbm, out_hbm,
            src_v, dst_v, gx0, gx1, gw0, gw1, agg_sh,
            sem_g0, sem_g1, sem_w0, sem_w1, sem_s0, sem_s1):
    c = lax.axis_index("c")
    s = lax.axis_index("s")
    wid = s * NC + c

    def _wslice(sl, j):
        return w_hbm.at[pl.ds(((wid * NSLAB + sl) * CPS + j) * C, C)]

    def _mul(gx, gw):
        half = FIL // 2

        def _row(i, carry):
            for g in range(half // 16):
                wv = plsc.bitcast(gw[i, pl.ds(16 * g, 16)], jnp.bfloat16)
                wa, wb = plsc.unpack(wv, format=plsc.PackFormat.INTERLEAVED)
                gx[i, pl.ds(16 * g, 16)] = wa * gx[i, pl.ds(16 * g, 16)]
                gx[i, pl.ds(half + 16 * g, 16)] = wb * gx[i, pl.ds(half + 16 * g, 16)]
            return carry

        lax.fori_loop(0, C, _row, 0)

    def _zero_buf(i, carry):
        for cc in range(FIL // 16):
            gx0[i, pl.ds(cc * 16, 16)] = jnp.zeros((16,), jnp.float32)
        return carry

    lax.fori_loop(0, RB, _zero_buf, 0)

    def _zero_stripe(t, carry):
        idx = s + t * NS

        @pl.when(idx < NRB)
        def _():
            pltpu.sync_copy(gx0, agg_sh.at[pl.ds(idx * RB, RB)])

        return carry

    lax.fori_loop(0, MAXT, _zero_stripe, 0)
    plsc.subcore_barrier()

    def _slab(sl, carry):
        pltpu.sync_copy(src_hbm.at[wid, sl], src_v)
        pltpu.sync_copy(dst_hbm.at[wid, sl], dst_v)
        pltpu.async_copy(x_hbm.at[src_v.at[0]], gx0, sem_g0)
        pltpu.async_copy(_wslice(sl, 0), gw0, sem_w0)

        def _pair(p, carry1):
            j0 = 2 * p
            j1 = 2 * p + 1

            @pl.when(p > 0)
            def _():
                pltpu.make_async_copy(gx1, agg_sh.at[dst_v.at[0]], sem_s1).wait()

            pltpu.async_copy(x_hbm.at[src_v.at[j1]], gx1, sem_g1)
            pltpu.async_copy(_wslice(sl, j1), gw1, sem_w1)

            pltpu.make_async_copy(x_hbm.at[src_v.at[j0]], gx0, sem_g0).wait()
            pltpu.make_async_copy(_wslice(sl, j0), gw0, sem_w0).wait()
            _mul(gx0, gw0)
            pltpu.async_copy(gx0, agg_sh.at[dst_v.at[j0]], sem_s0, add=True)

            pltpu.make_async_copy(x_hbm.at[src_v.at[j1]], gx1, sem_g1).wait()
            pltpu.make_async_copy(_wslice(sl, j1), gw1, sem_w1).wait()
            _mul(gx1, gw1)
            pltpu.async_copy(gx1, agg_sh.at[dst_v.at[j1]], sem_s1, add=True)

            @pl.when(p + 1 < PAIRS)
            def _():
                pltpu.make_async_copy(gx0, agg_sh.at[dst_v.at[0]], sem_s0).wait()
                pltpu.async_copy(x_hbm.at[src_v.at[j0 + 2]], gx0, sem_g0)
                pltpu.async_copy(_wslice(sl, j0 + 2), gw0, sem_w0)

            return carry1

        lax.fori_loop(0, PAIRS, _pair, 0)
        if CPS % 2:
            jt = CPS - 1
            pltpu.make_async_copy(gx0, agg_sh.at[dst_v.at[0]], sem_s0).wait()
            pltpu.async_copy(x_hbm.at[src_v.at[jt]], gx0, sem_g0)
            pltpu.async_copy(_wslice(sl, jt), gw0, sem_w0)
            pltpu.make_async_copy(x_hbm.at[src_v.at[jt]], gx0, sem_g0).wait()
            pltpu.make_async_copy(_wslice(sl, jt), gw0, sem_w0).wait()
            _mul(gx0, gw0)
            pltpu.async_copy(gx0, agg_sh.at[dst_v.at[jt]], sem_s0, add=True)
        pltpu.make_async_copy(gx0, agg_sh.at[dst_v.at[0]], sem_s0).wait()
        pltpu.make_async_copy(gx1, agg_sh.at[dst_v.at[0]], sem_s1).wait()
        return carry

    lax.fori_loop(0, NSLAB, _slab, 0)
    plsc.subcore_barrier()

    def _writeback(t, carry):
        idx = s + t * NS

        @pl.when(idx < NRB)
        def _():
            row = idx * RB
            pltpu.sync_copy(agg_sh.at[pl.ds(row, RB)], gx0)
            pltpu.sync_copy(gx0, out_hbm.at[c, pl.ds(row, RB)])

        return carry

    lax.fori_loop(0, MAXT, _writeback, 0)


def kernel(h, edge_index, dist_feat, fw1, fb1, fw2, fb2, aw1, ab1, aw2, ab2, ow1, ob1, ow2, ob2):
    src = edge_index[0].astype(jnp.int32).reshape(NW, NSLAB, CPS, C)
    dst = edge_index[1].astype(jnp.int32).reshape(NW, NSLAB, CPS, C)

    bf = jnp.bfloat16
    x = _mlp2(h.astype(bf), aw1.astype(bf), ab1.reshape(1, -1),
              aw2.astype(bf), ab2.reshape(1, -1), bm=2000, pack_out=False)
    w = _mlp2(dist_feat.astype(bf), fw1.astype(bf), fb1.reshape(1, -1),
              fw2.astype(bf), fb2.reshape(1, -1), bm=5000, pack_out=True)

    agg2 = _sc_agg(x, w, src, dst)

    return _out_mlp(h, agg2, ow1, ob1.reshape(1, -1), ow2, ob2.reshape(1, -1), bm=1000)


# speedup vs baseline: 1.4247x; 1.3083x over previous
import functools

import jax
import jax.numpy as jnp
from jax import lax
from jax.experimental import pallas as pl
from jax.experimental.pallas import tpu as pltpu
from jax.experimental.pallas import tpu_sc as plsc

N = 10000
E = 320000
HID = 128
FIL = 128
NG = 50

NC = 2
NS = 16
NW = NC * NS
EPW = E // NW
C = 80
NCH = EPW // C
NSLAB = 5
CPS = NCH // NSLAB
PAIRS = CPS // 2
RB = 80
NRB = N // RB
MAXT = (NRB + NS - 1) // NS


def _mlp2_body(pack_out, lhs_t, x_ref, w1_ref, b1_ref, w2_ref, b2_ref, o_ref):
    if lhs_t:
        t = jax.lax.dot_general(
            x_ref[...].astype(w1_ref.dtype), w1_ref[...],
            (((0,), (0,)), ((), ())), preferred_element_type=jnp.float32)
    else:
        t = jnp.dot(x_ref[...], w1_ref[...], preferred_element_type=jnp.float32)
    t = t + b1_ref[...]
    t = t * jax.lax.logistic(t)
    t = t.astype(w2_ref.dtype)
    o = jnp.dot(t, w2_ref[...], preferred_element_type=jnp.float32) + b2_ref[...]
    if not pack_out:
        o_ref[...] = o
        return
    f2 = o.shape[-1]
    a = jax.lax.bitcast_convert_type(o[:, : f2 // 2].astype(jnp.bfloat16), jnp.uint16)
    b = jax.lax.bitcast_convert_type(o[:, f2 // 2 :].astype(jnp.bfloat16), jnp.uint16)
    word = a.astype(jnp.uint32) | (b.astype(jnp.uint32) << 16)
    o_ref[...] = jax.lax.bitcast_convert_type(word, jnp.float32)


def _mlp2(x, w1, b1, w2, b2, bm, pack_out, lhs_t=False):
    if lhs_t:
        k, m = x.shape
        x_spec = pl.BlockSpec((k, bm), lambda i: (0, i))
    else:
        m, k = x.shape
        x_spec = pl.BlockSpec((bm, k), lambda i: (i, 0))
    f1 = w1.shape[1]
    f2 = w2.shape[1]
    fo = f2 // 2 if pack_out else f2
    return pl.pallas_call(
        functools.partial(_mlp2_body, pack_out, lhs_t),
        grid=(m // bm,),
        in_specs=[
            x_spec,
            pl.BlockSpec((k, f1), lambda i: (0, 0)),
            pl.BlockSpec((1, f1), lambda i: (0, 0)),
            pl.BlockSpec((f1, f2), lambda i: (0, 0)),
            pl.BlockSpec((1, f2), lambda i: (0, 0)),
        ],
        out_specs=pl.BlockSpec((bm, fo), lambda i: (i, 0)),
        out_shape=jax.ShapeDtypeStruct((m, fo), jnp.float32),
    )(x, w1, b1, w2, b2)


def _out_mlp_body(h_ref, agg_ref, w1_ref, b1_ref, w2_ref, b2_ref, o_ref):
    agg = agg_ref[0] + agg_ref[1]
    t = jnp.dot(agg, w1_ref[...], preferred_element_type=jnp.float32) + b1_ref[...]
    t = t * jax.lax.logistic(t)
    o_ref[...] = h_ref[...] + jnp.dot(t, w2_ref[...], preferred_element_type=jnp.float32) + b2_ref[...]


def _out_mlp(h, agg2, w1, b1, w2, b2, bm):
    m = h.shape[0]
    return pl.pallas_call(
        _out_mlp_body,
        grid=(m // bm,),
        in_specs=[
            pl.BlockSpec((bm, HID), lambda i: (i, 0)),
            pl.BlockSpec((NC, bm, FIL), lambda i: (0, i, 0)),
            pl.BlockSpec((FIL, HID), lambda i: (0, 0)),
            pl.BlockSpec((1, HID), lambda i: (0, 0)),
            pl.BlockSpec((HID, HID), lambda i: (0, 0)),
            pl.BlockSpec((1, HID), lambda i: (0, 0)),
        ],
        out_specs=pl.BlockSpec((bm, HID), lambda i: (i, 0)),
        out_shape=jax.ShapeDtypeStruct((m, HID), jnp.float32),
    )(h, agg2, w1, b1, w2, b2)


@functools.partial(
    pl.kernel,
    mesh=plsc.VectorSubcoreMesh(core_axis_name="c", subcore_axis_name="s"),
    compiler_params=pltpu.CompilerParams(needs_layout_passes=False),
    out_type=jax.ShapeDtypeStruct((NC, N, FIL), jnp.float32),
    scratch_types=[
        pltpu.VMEM((CPS, C), jnp.int32),
        pltpu.VMEM((CPS, C), jnp.int32),
        pltpu.VMEM((C, FIL), jnp.float32),
        pltpu.VMEM((C, FIL), jnp.float32),
        pltpu.VMEM((C, FIL // 2), jnp.float32),
        pltpu.VMEM((C, FIL // 2), jnp.float32),
        pltpu.VMEM_SHARED((N, FIL), jnp.float32),
        pltpu.SemaphoreType.DMA,
        pltpu.SemaphoreType.DMA,
        pltpu.SemaphoreType.DMA,
        pltpu.SemaphoreType.DMA,
        pltpu.SemaphoreType.DMA,
        pltpu.SemaphoreType.DMA,
    ],
)
def _sc_agg(x_hbm, w_hbm, src_hbm, dst_hbm, out_hbm,
            src_v, dst_v, gx0, gx1, gw0, gw1, agg_sh,
            sem_g0, sem_g1, sem_w0, sem_w1, sem_s0, sem_s1):
    c = lax.axis_index("c")
    s = lax.axis_index("s")
    wid = s * NC + c

    def _wslice(sl, j):
        return w_hbm.at[pl.ds(((wid * NSLAB + sl) * CPS + j) * C, C)]

    def _mul(gx, gw):
        half = FIL // 2

        def _row(i, carry):
            for g in range(half // 16):
                wv = plsc.bitcast(gw[i, pl.ds(16 * g, 16)], jnp.bfloat16)
                wa, wb = plsc.unpack(wv, format=plsc.PackFormat.INTERLEAVED)
                gx[i, pl.ds(16 * g, 16)] = wa * gx[i, pl.ds(16 * g, 16)]
                gx[i, pl.ds(half + 16 * g, 16)] = wb * gx[i, pl.ds(half + 16 * g, 16)]
            return carry

        lax.fori_loop(0, C, _row, 0)

    def _zero_buf(i, carry):
        for cc in range(FIL // 16):
            gx0[i, pl.ds(cc * 16, 16)] = jnp.zeros((16,), jnp.float32)
        return carry

    lax.fori_loop(0, RB, _zero_buf, 0)

    def _zero_stripe(t, carry):
        idx = s + t * NS

        @pl.when(idx < NRB)
        def _():
            pltpu.sync_copy(gx0, agg_sh.at[pl.ds(idx * RB, RB)])

        return carry

    lax.fori_loop(0, MAXT, _zero_stripe, 0)
    plsc.subcore_barrier()

    def _slab(sl, carry):
        pltpu.sync_copy(src_hbm.at[wid, sl], src_v)
        pltpu.sync_copy(dst_hbm.at[wid, sl], dst_v)
        pltpu.async_copy(x_hbm.at[src_v.at[0]], gx0, sem_g0)
        pltpu.async_copy(_wslice(sl, 0), gw0, sem_w0)

        def _pair(p, carry1):
            j0 = 2 * p
            j1 = 2 * p + 1

            @pl.when(p > 0)
            def _():
                pltpu.make_async_copy(gx1, agg_sh.at[dst_v.at[0]], sem_s1).wait()

            pltpu.async_copy(x_hbm.at[src_v.at[j1]], gx1, sem_g1)
            pltpu.async_copy(_wslice(sl, j1), gw1, sem_w1)

            pltpu.make_async_copy(x_hbm.at[src_v.at[j0]], gx0, sem_g0).wait()
            pltpu.make_async_copy(_wslice(sl, j0), gw0, sem_w0).wait()
            _mul(gx0, gw0)
            pltpu.async_copy(gx0, agg_sh.at[dst_v.at[j0]], sem_s0, add=True)

            pltpu.make_async_copy(x_hbm.at[src_v.at[j1]], gx1, sem_g1).wait()
            pltpu.make_async_copy(_wslice(sl, j1), gw1, sem_w1).wait()
            _mul(gx1, gw1)
            pltpu.async_copy(gx1, agg_sh.at[dst_v.at[j1]], sem_s1, add=True)

            @pl.when(p + 1 < PAIRS)
            def _():
                pltpu.make_async_copy(gx0, agg_sh.at[dst_v.at[0]], sem_s0).wait()
                pltpu.async_copy(x_hbm.at[src_v.at[j0 + 2]], gx0, sem_g0)
                pltpu.async_copy(_wslice(sl, j0 + 2), gw0, sem_w0)

            return carry1

        lax.fori_loop(0, PAIRS, _pair, 0)
        if CPS % 2:
            jt = CPS - 1
            pltpu.make_async_copy(gx0, agg_sh.at[dst_v.at[0]], sem_s0).wait()
            pltpu.async_copy(x_hbm.at[src_v.at[jt]], gx0, sem_g0)
            pltpu.async_copy(_wslice(sl, jt), gw0, sem_w0)
            pltpu.make_async_copy(x_hbm.at[src_v.at[jt]], gx0, sem_g0).wait()
            pltpu.make_async_copy(_wslice(sl, jt), gw0, sem_w0).wait()
            _mul(gx0, gw0)
            pltpu.async_copy(gx0, agg_sh.at[dst_v.at[jt]], sem_s0, add=True)
        pltpu.make_async_copy(gx0, agg_sh.at[dst_v.at[0]], sem_s0).wait()
        pltpu.make_async_copy(gx1, agg_sh.at[dst_v.at[0]], sem_s1).wait()
        return carry

    lax.fori_loop(0, NSLAB, _slab, 0)
    plsc.subcore_barrier()

    def _writeback(t, carry):
        idx = s + t * NS

        @pl.when(idx < NRB)
        def _():
            row = idx * RB
            pltpu.sync_copy(agg_sh.at[pl.ds(row, RB)], gx0)
            pltpu.sync_copy(gx0, out_hbm.at[c, pl.ds(row, RB)])

        return carry

    lax.fori_loop(0, MAXT, _writeback, 0)


def kernel(h, edge_index, dist_feat, fw1, fb1, fw2, fb2, aw1, ab1, aw2, ab2, ow1, ob1, ow2, ob2):
    src = edge_index[0].astype(jnp.int32).reshape(NW, NSLAB, CPS, C)
    dst = edge_index[1].astype(jnp.int32).reshape(NW, NSLAB, CPS, C)

    bf = jnp.bfloat16
    x = _mlp2(h.astype(bf), aw1.astype(bf), ab1.reshape(1, -1),
              aw2.astype(bf), ab2.reshape(1, -1), bm=2000, pack_out=False)
    w = _mlp2(dist_feat.T, fw1.astype(bf), fb1.reshape(1, -1),
              fw2.astype(bf), fb2.reshape(1, -1), bm=6400, pack_out=True,
              lhs_t=True)

    agg2 = _sc_agg(x, w, src, dst)

    return _out_mlp(h, agg2, ow1, ob1.reshape(1, -1), ow2, ob2.reshape(1, -1), bm=1000)


# speedup vs baseline: 1.4350x; 1.0073x over previous
import functools

import jax
import jax.numpy as jnp
from jax import lax
from jax.experimental import pallas as pl
from jax.experimental.pallas import tpu as pltpu
from jax.experimental.pallas import tpu_sc as plsc

N = 10000
E = 320000
HID = 128
FIL = 128
NG = 50

NC = 2
NS = 16
NW = NC * NS
EPW = E // NW
C = 80
NCH = EPW // C
NSLAB = 5
CPS = NCH // NSLAB
PAIRS = CPS // 2
RB = 80
NRB = N // RB
MAXT = (NRB + NS - 1) // NS


def _mlp2_body(pack_out, lhs_t, x_ref, w1_ref, b1_ref, w2_ref, b2_ref, o_ref):
    if lhs_t:
        t = jax.lax.dot_general(
            x_ref[...].astype(w1_ref.dtype), w1_ref[...],
            (((0,), (0,)), ((), ())), preferred_element_type=jnp.float32)
    else:
        t = jnp.dot(x_ref[...], w1_ref[...], preferred_element_type=jnp.float32)
    t = t + b1_ref[...]
    t = (0.5 * t) * (jnp.tanh(t * 0.5) + 1.0)
    t = t.astype(w2_ref.dtype)
    o = jnp.dot(t, w2_ref[...], preferred_element_type=jnp.float32) + b2_ref[...]
    if not pack_out:
        o_ref[...] = o
        return
    f2 = o.shape[-1]
    a = jax.lax.bitcast_convert_type(o[:, : f2 // 2].astype(jnp.bfloat16), jnp.uint16)
    b = jax.lax.bitcast_convert_type(o[:, f2 // 2 :].astype(jnp.bfloat16), jnp.uint16)
    word = a.astype(jnp.uint32) | (b.astype(jnp.uint32) << 16)
    o_ref[...] = jax.lax.bitcast_convert_type(word, jnp.float32)


def _mlp2(x, w1, b1, w2, b2, bm, pack_out, lhs_t=False):
    if lhs_t:
        k, m = x.shape
        x_spec = pl.BlockSpec((k, bm), lambda i: (0, i))
    else:
        m, k = x.shape
        x_spec = pl.BlockSpec((bm, k), lambda i: (i, 0))
    f1 = w1.shape[1]
    f2 = w2.shape[1]
    fo = f2 // 2 if pack_out else f2
    return pl.pallas_call(
        functools.partial(_mlp2_body, pack_out, lhs_t),
        grid=(m // bm,),
        in_specs=[
            x_spec,
            pl.BlockSpec((k, f1), lambda i: (0, 0)),
            pl.BlockSpec((1, f1), lambda i: (0, 0)),
            pl.BlockSpec((f1, f2), lambda i: (0, 0)),
            pl.BlockSpec((1, f2), lambda i: (0, 0)),
        ],
        out_specs=pl.BlockSpec((bm, fo), lambda i: (i, 0)),
        out_shape=jax.ShapeDtypeStruct((m, fo), jnp.float32),
    )(x, w1, b1, w2, b2)


def _out_mlp_body(h_ref, agg_ref, w1_ref, b1_ref, w2_ref, b2_ref, o_ref):
    agg = agg_ref[0] + agg_ref[1]
    t = jnp.dot(agg, w1_ref[...], preferred_element_type=jnp.float32) + b1_ref[...]
    t = (0.5 * t) * (jnp.tanh(t * 0.5) + 1.0)
    o_ref[...] = h_ref[...] + jnp.dot(t, w2_ref[...], preferred_element_type=jnp.float32) + b2_ref[...]


def _out_mlp(h, agg2, w1, b1, w2, b2, bm):
    m = h.shape[0]
    return pl.pallas_call(
        _out_mlp_body,
        grid=(m // bm,),
        in_specs=[
            pl.BlockSpec((bm, HID), lambda i: (i, 0)),
            pl.BlockSpec((NC, bm, FIL), lambda i: (0, i, 0)),
            pl.BlockSpec((FIL, HID), lambda i: (0, 0)),
            pl.BlockSpec((1, HID), lambda i: (0, 0)),
            pl.BlockSpec((HID, HID), lambda i: (0, 0)),
            pl.BlockSpec((1, HID), lambda i: (0, 0)),
        ],
        out_specs=pl.BlockSpec((bm, HID), lambda i: (i, 0)),
        out_shape=jax.ShapeDtypeStruct((m, HID), jnp.float32),
    )(h, agg2, w1, b1, w2, b2)


@functools.partial(
    pl.kernel,
    mesh=plsc.VectorSubcoreMesh(core_axis_name="c", subcore_axis_name="s"),
    compiler_params=pltpu.CompilerParams(needs_layout_passes=False),
    out_type=jax.ShapeDtypeStruct((NC, N, FIL), jnp.float32),
    scratch_types=[
        pltpu.VMEM((CPS, C), jnp.int32),
        pltpu.VMEM((CPS, C), jnp.int32),
        pltpu.VMEM((C, FIL), jnp.float32),
        pltpu.VMEM((C, FIL), jnp.float32),
        pltpu.VMEM((C, FIL // 2), jnp.float32),
        pltpu.VMEM((C, FIL // 2), jnp.float32),
        pltpu.VMEM_SHARED((N, FIL), jnp.float32),
        pltpu.SemaphoreType.DMA,
        pltpu.SemaphoreType.DMA,
        pltpu.SemaphoreType.DMA,
        pltpu.SemaphoreType.DMA,
        pltpu.SemaphoreType.DMA,
        pltpu.SemaphoreType.DMA,
    ],
)
def _sc_agg(x_hbm, w_hbm, src_hbm, dst_hbm, out_hbm,
            src_v, dst_v, gx0, gx1, gw0, gw1, agg_sh,
            sem_g0, sem_g1, sem_w0, sem_w1, sem_s0, sem_s1):
    c = lax.axis_index("c")
    s = lax.axis_index("s")
    wid = s * NC + c

    def _wslice(sl, j):
        return w_hbm.at[pl.ds(((wid * NSLAB + sl) * CPS + j) * C, C)]

    def _mul(gx, gw):
        half = FIL // 2

        def _row(i, carry):
            for g in range(half // 16):
                wv = plsc.bitcast(gw[i, pl.ds(16 * g, 16)], jnp.bfloat16)
                wa, wb = plsc.unpack(wv, format=plsc.PackFormat.INTERLEAVED)
                gx[i, pl.ds(16 * g, 16)] = wa * gx[i, pl.ds(16 * g, 16)]
                gx[i, pl.ds(half + 16 * g, 16)] = wb * gx[i, pl.ds(half + 16 * g, 16)]
            return carry

        lax.fori_loop(0, C, _row, 0)

    def _zero_buf(i, carry):
        for cc in range(FIL // 16):
            gx0[i, pl.ds(cc * 16, 16)] = jnp.zeros((16,), jnp.float32)
        return carry

    lax.fori_loop(0, RB, _zero_buf, 0)

    def _zero_stripe(t, carry):
        idx = s + t * NS

        @pl.when(idx < NRB)
        def _():
            pltpu.sync_copy(gx0, agg_sh.at[pl.ds(idx * RB, RB)])

        return carry

    lax.fori_loop(0, MAXT, _zero_stripe, 0)
    plsc.subcore_barrier()

    def _slab(sl, carry):
        pltpu.sync_copy(src_hbm.at[wid, sl], src_v)
        pltpu.sync_copy(dst_hbm.at[wid, sl], dst_v)
        pltpu.async_copy(x_hbm.at[src_v.at[0]], gx0, sem_g0)
        pltpu.async_copy(_wslice(sl, 0), gw0, sem_w0)

        def _pair(p, carry1):
            j0 = 2 * p
            j1 = 2 * p + 1

            @pl.when(p > 0)
            def _():
                pltpu.make_async_copy(gx1, agg_sh.at[dst_v.at[0]], sem_s1).wait()

            pltpu.async_copy(x_hbm.at[src_v.at[j1]], gx1, sem_g1)
            pltpu.async_copy(_wslice(sl, j1), gw1, sem_w1)

            pltpu.make_async_copy(x_hbm.at[src_v.at[j0]], gx0, sem_g0).wait()
            pltpu.make_async_copy(_wslice(sl, j0), gw0, sem_w0).wait()
            _mul(gx0, gw0)
            pltpu.async_copy(gx0, agg_sh.at[dst_v.at[j0]], sem_s0, add=True)

            pltpu.make_async_copy(x_hbm.at[src_v.at[j1]], gx1, sem_g1).wait()
            pltpu.make_async_copy(_wslice(sl, j1), gw1, sem_w1).wait()
            _mul(gx1, gw1)
            pltpu.async_copy(gx1, agg_sh.at[dst_v.at[j1]], sem_s1, add=True)

            @pl.when(p + 1 < PAIRS)
            def _():
                pltpu.make_async_copy(gx0, agg_sh.at[dst_v.at[0]], sem_s0).wait()
                pltpu.async_copy(x_hbm.at[src_v.at[j0 + 2]], gx0, sem_g0)
                pltpu.async_copy(_wslice(sl, j0 + 2), gw0, sem_w0)

            return carry1

        lax.fori_loop(0, PAIRS, _pair, 0)
        if CPS % 2:
            jt = CPS - 1
            pltpu.make_async_copy(gx0, agg_sh.at[dst_v.at[0]], sem_s0).wait()
            pltpu.async_copy(x_hbm.at[src_v.at[jt]], gx0, sem_g0)
            pltpu.async_copy(_wslice(sl, jt), gw0, sem_w0)
            pltpu.make_async_copy(x_hbm.at[src_v.at[jt]], gx0, sem_g0).wait()
            pltpu.make_async_copy(_wslice(sl, jt), gw0, sem_w0).wait()
            _mul(gx0, gw0)
            pltpu.async_copy(gx0, agg_sh.at[dst_v.at[jt]], sem_s0, add=True)
        pltpu.make_async_copy(gx0, agg_sh.at[dst_v.at[0]], sem_s0).wait()
        pltpu.make_async_copy(gx1, agg_sh.at[dst_v.at[0]], sem_s1).wait()
        return carry

    lax.fori_loop(0, NSLAB, _slab, 0)
    plsc.subcore_barrier()

    def _writeback(t, carry):
        idx = s + t * NS

        @pl.when(idx < NRB)
        def _():
            row = idx * RB
            pltpu.sync_copy(agg_sh.at[pl.ds(row, RB)], gx0)
            pltpu.sync_copy(gx0, out_hbm.at[c, pl.ds(row, RB)])

        return carry

    lax.fori_loop(0, MAXT, _writeback, 0)


def kernel(h, edge_index, dist_feat, fw1, fb1, fw2, fb2, aw1, ab1, aw2, ab2, ow1, ob1, ow2, ob2):
    src = edge_index[0].astype(jnp.int32).reshape(NW, NSLAB, CPS, C)
    dst = edge_index[1].astype(jnp.int32).reshape(NW, NSLAB, CPS, C)

    bf = jnp.bfloat16
    x = _mlp2(h.astype(bf), aw1.astype(bf), ab1.reshape(1, -1),
              aw2.astype(bf), ab2.reshape(1, -1), bm=2000, pack_out=False)
    w = _mlp2(dist_feat.T, fw1.astype(bf), fb1.reshape(1, -1),
              fw2.astype(bf), fb2.reshape(1, -1), bm=6400, pack_out=True,
              lhs_t=True)

    agg2 = _sc_agg(x, w, src, dst)

    return _out_mlp(h, agg2, ow1, ob1.reshape(1, -1), ow2, ob2.reshape(1, -1), bm=1000)


# speedup vs baseline: 1.5122x; 1.0538x over previous
import functools

import jax
import jax.numpy as jnp
from jax import lax
from jax.experimental import pallas as pl
from jax.experimental.pallas import tpu as pltpu
from jax.experimental.pallas import tpu_sc as plsc

N = 10000
E = 320000
HID = 128
FIL = 128
NG = 50

NC = 2
NS = 16
NW = NC * NS
EPW = E // NW
C = 40
NCH = EPW // C
NSLAB = 5
CPS = NCH // NSLAB
RB = 40
NRB = N // RB
MAXT = (NRB + NS - 1) // NS


def _mlp2_body(pack_out, lhs_t, x_ref, w1_ref, b1_ref, w2_ref, b2_ref, o_ref):
    if lhs_t:
        t = jax.lax.dot_general(
            x_ref[...].astype(w1_ref.dtype), w1_ref[...],
            (((0,), (0,)), ((), ())), preferred_element_type=jnp.float32)
    else:
        t = jnp.dot(x_ref[...], w1_ref[...], preferred_element_type=jnp.float32)
    t = t + b1_ref[...]
    t = (0.5 * t) * (jnp.tanh(t * 0.5) + 1.0)
    t = t.astype(w2_ref.dtype)
    o = jnp.dot(t, w2_ref[...], preferred_element_type=jnp.float32) + b2_ref[...]
    if not pack_out:
        o_ref[...] = o
        return
    f2 = o.shape[-1]
    a = jax.lax.bitcast_convert_type(o[:, : f2 // 2].astype(jnp.bfloat16), jnp.uint16)
    b = jax.lax.bitcast_convert_type(o[:, f2 // 2 :].astype(jnp.bfloat16), jnp.uint16)
    word = a.astype(jnp.uint32) | (b.astype(jnp.uint32) << 16)
    o_ref[...] = jax.lax.bitcast_convert_type(word, jnp.float32)


def _mlp2(x, w1, b1, w2, b2, bm, pack_out, lhs_t=False):
    if lhs_t:
        k, m = x.shape
        x_spec = pl.BlockSpec((k, bm), lambda i: (0, i))
    else:
        m, k = x.shape
        x_spec = pl.BlockSpec((bm, k), lambda i: (i, 0))
    f1 = w1.shape[1]
    f2 = w2.shape[1]
    fo = f2 // 2 if pack_out else f2
    return pl.pallas_call(
        functools.partial(_mlp2_body, pack_out, lhs_t),
        grid=(m // bm,),
        in_specs=[
            x_spec,
            pl.BlockSpec((k, f1), lambda i: (0, 0)),
            pl.BlockSpec((1, f1), lambda i: (0, 0)),
            pl.BlockSpec((f1, f2), lambda i: (0, 0)),
            pl.BlockSpec((1, f2), lambda i: (0, 0)),
        ],
        out_specs=pl.BlockSpec((bm, fo), lambda i: (i, 0)),
        out_shape=jax.ShapeDtypeStruct((m, fo), jnp.float32),
    )(x, w1, b1, w2, b2)


def _out_mlp_body(h_ref, agg_ref, w1_ref, b1_ref, w2_ref, b2_ref, o_ref):
    agg = agg_ref[0] + agg_ref[1]
    t = jnp.dot(agg, w1_ref[...], preferred_element_type=jnp.float32) + b1_ref[...]
    t = (0.5 * t) * (jnp.tanh(t * 0.5) + 1.0)
    o_ref[...] = h_ref[...] + jnp.dot(t, w2_ref[...], preferred_element_type=jnp.float32) + b2_ref[...]


def _out_mlp(h, agg2, w1, b1, w2, b2, bm):
    m = h.shape[0]
    return pl.pallas_call(
        _out_mlp_body,
        grid=(m // bm,),
        in_specs=[
            pl.BlockSpec((bm, HID), lambda i: (i, 0)),
            pl.BlockSpec((NC, bm, FIL), lambda i: (0, i, 0)),
            pl.BlockSpec((FIL, HID), lambda i: (0, 0)),
            pl.BlockSpec((1, HID), lambda i: (0, 0)),
            pl.BlockSpec((HID, HID), lambda i: (0, 0)),
            pl.BlockSpec((1, HID), lambda i: (0, 0)),
        ],
        out_specs=pl.BlockSpec((bm, HID), lambda i: (i, 0)),
        out_shape=jax.ShapeDtypeStruct((m, HID), jnp.float32),
    )(h, agg2, w1, b1, w2, b2)


@functools.partial(
    pl.kernel,
    mesh=plsc.VectorSubcoreMesh(core_axis_name="c", subcore_axis_name="s"),
    compiler_params=pltpu.CompilerParams(needs_layout_passes=False),
    out_type=jax.ShapeDtypeStruct((NC, N, FIL), jnp.float32),
    scratch_types=[
        pltpu.VMEM((CPS, C), jnp.int32),
        pltpu.VMEM((CPS, C), jnp.int32),
        pltpu.VMEM((C, FIL), jnp.float32),
        pltpu.VMEM((C, FIL), jnp.float32),
        pltpu.VMEM((C, FIL), jnp.float32),
        pltpu.VMEM((C, FIL // 2), jnp.float32),
        pltpu.VMEM((C, FIL // 2), jnp.float32),
        pltpu.VMEM((C, FIL // 2), jnp.float32),
        pltpu.VMEM_SHARED((N, FIL), jnp.float32),
        pltpu.SemaphoreType.DMA,
        pltpu.SemaphoreType.DMA,
        pltpu.SemaphoreType.DMA,
        pltpu.SemaphoreType.DMA,
        pltpu.SemaphoreType.DMA,
        pltpu.SemaphoreType.DMA,
        pltpu.SemaphoreType.DMA,
        pltpu.SemaphoreType.DMA,
        pltpu.SemaphoreType.DMA,
    ],
)
def _sc_agg(x_hbm, w_hbm, src_hbm, dst_hbm, out_hbm,
            src_v, dst_v, gx0, gx1, gx2, gw0, gw1, gw2, agg_sh,
            sem_g0, sem_g1, sem_g2, sem_w0, sem_w1, sem_w2,
            sem_s0, sem_s1, sem_s2):
    c = lax.axis_index("c")
    s = lax.axis_index("s")
    wid = s * NC + c

    def _wslice(sl, j):
        return w_hbm.at[pl.ds(((wid * NSLAB + sl) * CPS + j) * C, C)]

    def _mul(gx, gw):
        half = FIL // 2

        def _row(i, carry):
            for g in range(half // 16):
                wv = plsc.bitcast(gw[i, pl.ds(16 * g, 16)], jnp.bfloat16)
                wa, wb = plsc.unpack(wv, format=plsc.PackFormat.INTERLEAVED)
                gx[i, pl.ds(16 * g, 16)] = wa * gx[i, pl.ds(16 * g, 16)]
                gx[i, pl.ds(half + 16 * g, 16)] = wb * gx[i, pl.ds(half + 16 * g, 16)]
            return carry

        lax.fori_loop(0, C, _row, 0)

    def _zero_buf(i, carry):
        for cc in range(FIL // 16):
            gx0[i, pl.ds(cc * 16, 16)] = jnp.zeros((16,), jnp.float32)
        return carry

    lax.fori_loop(0, RB, _zero_buf, 0)

    def _zero_stripe(t, carry):
        idx = s + t * NS

        @pl.when(idx < NRB)
        def _():
            pltpu.sync_copy(gx0, agg_sh.at[pl.ds(idx * RB, RB)])

        return carry

    lax.fori_loop(0, MAXT, _zero_stripe, 0)
    plsc.subcore_barrier()

    SETS = ((gx0, gw0, sem_g0, sem_w0, sem_s0),
            (gx1, gw1, sem_g1, sem_w1, sem_s1),
            (gx2, gw2, sem_g2, sem_w2, sem_s2))

    def _slab(sl, carry):
        pltpu.sync_copy(src_hbm.at[wid, sl], src_v)
        pltpu.sync_copy(dst_hbm.at[wid, sl], dst_v)
        for r in range(2):
            gxr, gwr, gr, wr_, _ = SETS[r]
            pltpu.async_copy(x_hbm.at[src_v.at[r]], gxr, gr)
            pltpu.async_copy(_wslice(sl, r), gwr, wr_)

        def _chunk(j, carry1):
            for r in range(3):

                @pl.when(j % 3 == r)
                def _(r=r):
                    gxr, gwr, gr, wr_, sr = SETS[r]
                    gxn, gwn, gn, wn_, sn = SETS[(r + 2) % 3]
                    pltpu.make_async_copy(x_hbm.at[src_v.at[j]], gxr, gr).wait()
                    pltpu.make_async_copy(_wslice(sl, j), gwr, wr_).wait()
                    _mul(gxr, gwr)
                    pltpu.async_copy(gxr, agg_sh.at[dst_v.at[j]], sr, add=True)

                    @pl.when(j + 2 < CPS)
                    def _():
                        @pl.when(j >= 1)
                        def _():
                            pltpu.make_async_copy(
                                gxn, agg_sh.at[dst_v.at[0]], sn).wait()

                        pltpu.async_copy(x_hbm.at[src_v.at[j + 2]], gxn, gn)
                        pltpu.async_copy(_wslice(sl, j + 2), gwn, wn_)

            return carry1

        lax.fori_loop(0, CPS, _chunk, 0)
        for r in range(3):
            gxr, _, _, _, sr = SETS[r]
            pltpu.make_async_copy(gxr, agg_sh.at[dst_v.at[0]], sr).wait()
        return carry

    lax.fori_loop(0, NSLAB, _slab, 0)
    plsc.subcore_barrier()

    def _writeback(t, carry):
        idx = s + t * NS

        @pl.when(idx < NRB)
        def _():
            row = idx * RB
            pltpu.sync_copy(agg_sh.at[pl.ds(row, RB)], gx0)
            pltpu.sync_copy(gx0, out_hbm.at[c, pl.ds(row, RB)])

        return carry

    lax.fori_loop(0, MAXT, _writeback, 0)


def kernel(h, edge_index, dist_feat, fw1, fb1, fw2, fb2, aw1, ab1, aw2, ab2, ow1, ob1, ow2, ob2):
    src = edge_index[0].astype(jnp.int32).reshape(NW, NSLAB, CPS, C)
    dst = edge_index[1].astype(jnp.int32).reshape(NW, NSLAB, CPS, C)

    bf = jnp.bfloat16
    x = _mlp2(h.astype(bf), aw1.astype(bf), ab1.reshape(1, -1),
              aw2.astype(bf), ab2.reshape(1, -1), bm=2000, pack_out=False)
    w = _mlp2(dist_feat.T, fw1.astype(bf), fb1.reshape(1, -1),
              fw2.astype(bf), fb2.reshape(1, -1), bm=6400, pack_out=True,
              lhs_t=True)

    agg2 = _sc_agg(x, w, src, dst)

    return _out_mlp(h, agg2, ow1, ob1.reshape(1, -1), ow2, ob2.reshape(1, -1), bm=1000)
